# baseline (speedup 1.0000x reference)
import functools

import jax
import jax.numpy as jnp
from jax import lax
from jax.experimental import pallas as pl
from jax.experimental.pallas import tpu as pltpu
from jax.experimental.pallas import tpu_sc as plsc

_info = plsc.get_sparse_core_info()
_NC, _NS = _info.num_cores, _info.num_subcores
_NW = _NC * _NS

_S = 16384
_P = 50
_D = 64
_SPW = _S // _NW
_SB = 128
_PB = 2
_CR = _PB * _SB
_NP = _P // _PB
_NCHUNK = (_SPW // _SB) * _NP


def _emb_call():
    mesh = plsc.VectorSubcoreMesh(core_axis_name="c", subcore_axis_name="s")

    @functools.partial(
        pl.kernel,
        mesh=mesh,
        out_type=jax.ShapeDtypeStruct((_P, 8, _S // 128, 8, 128), jnp.float32),
        scratch_types=[
            pltpu.VMEM((_SPW * _P,), jnp.int32),
            [pltpu.VMEM((_CR,), jnp.int32)] * 2,
            [pltpu.VMEM((_CR, _D), jnp.float32)] * 2,
            [pltpu.VMEM((_PB, 8, 1, 8, _SB + 1), jnp.float32)] * 2,
            [pltpu.SemaphoreType.DMA] * 2,
            [pltpu.SemaphoreType.DMA] * 2,
        ],
        compiler_params=pltpu.CompilerParams(
            use_tc_tiling_on_sc=False, needs_layout_passes=False
        ),
    )
    def emb(w_hbm, idx_hbm, out_hbm, idxall, sels, rows, tbufs, gsems, ssems):
        wid = lax.axis_index("s") * _NC + lax.axis_index("c")
        base_b = wid * (_SPW * _P)
        iota16 = lax.iota(jnp.int32, 16)
        iota50 = iota16 * _P

        pltpu.sync_copy(idx_hbm.at[pl.ds(base_b, _SPW * _P)], idxall)

        def gather_desc(slot):
            return pltpu.make_async_copy(
                w_hbm.at[sels[slot]], rows[slot], gsems[slot]
            )

        def scatter_desc(slot, p_blk, p0, c):
            return pltpu.make_async_copy(
                tbufs[slot].at[pl.ds(p_blk, 1), :, :, :, pl.ds(0, _SB)],
                out_hbm.at[pl.ds(p0 + p_blk, 1), :, pl.ds(c, 1), :, :],
                ssems[slot],
            )

        def build_sel(k, slot):
            kd = k // _NP
            km = k % _NP
            base = kd * (_SB * _P) + km * _PB
            for p_blk in range(_PB):
                for q in range(8):
                    ids = iota50 + (base + p_blk + (16 * q) * _P)
                    v = plsc.load_gather(idxall, [ids])
                    sels[slot][pl.ds(p_blk * _SB + 16 * q, 16)] = v

        rvecs = [iota16 // 8 + (16 * bi) // 8 for bi in range(4)]
        uvec = iota16 % 8
        zvec = jnp.zeros((16,), jnp.int32)
        pvecs = [jnp.full((16,), p_blk, jnp.int32) for p_blk in range(_PB)]

        def transpose_chunk(slot):
            @plsc.parallel_loop(0, _SB, unroll=8)
            def body(l):
                lvec = jnp.broadcast_to(l, (16,)).astype(jnp.int32)
                for p_blk in range(_PB):
                    r = p_blk * _SB + l
                    for bi in range(4):
                        v = rows[slot][r, pl.ds(16 * bi, 16)]
                        plsc.store_scatter(
                            tbufs[slot],
                            [pvecs[p_blk], rvecs[bi], zvec, uvec, lvec],
                            v,
                        )

        def process(k, slot):
            gather_desc(slot).wait()
            transpose_chunk(slot)
            kd = k // _NP
            km = k % _NP
            c = wid * (_SPW // _SB) + kd
            p0 = km * _PB
            for p_blk in range(_PB):
                scatter_desc(slot, p_blk, p0, c).start()
            knxt = jnp.minimum(k + 1, _NCHUNK - 1)
            build_sel(knxt, 1 - slot)
            gather_desc(1 - slot).start()
            for p_blk in range(_PB):
                scatter_desc(slot, p_blk, 0, 0).wait()

        build_sel(0, 0)
        gather_desc(0).start()

        def body(t, carry):
            process(2 * t, 0)
            process(2 * t + 1, 1)
            return carry

        lax.fori_loop(0, _NCHUNK // 2, body, 0)
        gather_desc(0).wait()

    return emb


def kernel(weight, x):
    out5 = _emb_call()(weight, x.reshape(_S * _P).astype(jnp.int32))
    return out5.transpose(2, 4, 0, 1, 3).reshape(_S, _P, _D)


# speedup vs baseline: 1.0209x; 1.0209x over previous
import functools

import jax
import jax.numpy as jnp
from jax import lax
from jax.experimental import pallas as pl
from jax.experimental.pallas import tpu as pltpu
from jax.experimental.pallas import tpu_sc as plsc

_info = plsc.get_sparse_core_info()
_NC, _NS = _info.num_cores, _info.num_subcores
_NW = _NC * _NS

_S = 16384
_P = 50
_D = 64
_SPW = _S // _NW
_SB = 128
_PB = 2
_CR = _PB * _SB
_NP = _P // _PB
_NCHUNK = (_SPW // _SB) * _NP


def _emb_call():
    mesh = plsc.VectorSubcoreMesh(core_axis_name="c", subcore_axis_name="s")

    @functools.partial(
        pl.kernel,
        mesh=mesh,
        out_type=jax.ShapeDtypeStruct((_P, 8, _S // 128, 8, 128), jnp.float32),
        scratch_types=[
            pltpu.VMEM((_SPW * _P,), jnp.int32),
            [pltpu.VMEM((_CR,), jnp.int32)] * 2,
            [pltpu.VMEM((_CR, _D), jnp.float32)] * 2,
            [pltpu.VMEM((_PB, 8, 1, 8, _SB + 1), jnp.float32)] * 2,
            [pltpu.SemaphoreType.DMA] * 2,
            [pltpu.SemaphoreType.DMA] * 2,
        ],
        compiler_params=pltpu.CompilerParams(
            use_tc_tiling_on_sc=False, needs_layout_passes=False
        ),
    )
    def emb(w_hbm, idx_hbm, out_hbm, idxall, sels, rows, tbufs, gsems, ssems):
        wid = lax.axis_index("s") * _NC + lax.axis_index("c")
        base_b = wid * (_SPW * _P)
        iota16 = lax.iota(jnp.int32, 16)
        iota50 = iota16 * _P

        pltpu.sync_copy(idx_hbm.at[pl.ds(base_b, _SPW * _P)], idxall)

        def gather_desc(slot):
            return pltpu.make_async_copy(
                w_hbm.at[sels[slot]], rows[slot], gsems[slot]
            )

        def scatter_desc(slot, p_blk, p0, c):
            return pltpu.make_async_copy(
                tbufs[slot].at[pl.ds(p_blk, 1), :, :, :, pl.ds(0, _SB)],
                out_hbm.at[pl.ds(p0 + p_blk, 1), :, pl.ds(c, 1), :, :],
                ssems[slot],
            )

        def build_sel(k, slot):
            kd = k // _NP
            km = k % _NP
            base = kd * (_SB * _P) + km * _PB
            for p_blk in range(_PB):
                for q in range(8):
                    ids = iota50 + (base + p_blk + (16 * q) * _P)
                    v = plsc.load_gather(idxall, [ids])
                    sels[slot][pl.ds(p_blk * _SB + 16 * q, 16)] = v

        rvecs = [iota16 // 8 + (16 * bi) // 8 for bi in range(4)]
        uvec = iota16 % 8
        zvec = jnp.zeros((16,), jnp.int32)
        pvecs = [jnp.full((16,), p_blk, jnp.int32) for p_blk in range(_PB)]

        def transpose_chunk(slot):
            @plsc.parallel_loop(0, _SB, unroll=16)
            def body(l):
                lvec = jnp.broadcast_to(l, (16,)).astype(jnp.int32)
                for p_blk in range(_PB):
                    r = p_blk * _SB + l
                    for bi in range(4):
                        v = rows[slot][r, pl.ds(16 * bi, 16)]
                        plsc.store_scatter(
                            tbufs[slot],
                            [pvecs[p_blk], rvecs[bi], zvec, uvec, lvec],
                            v,
                        )

        def process(k, slot, wait_prev_scatter, issue_next):
            gather_desc(slot).wait()
            if wait_prev_scatter:
                for p_blk in range(_PB):
                    scatter_desc(slot, p_blk, 0, 0).wait()
            transpose_chunk(slot)
            kd = k // _NP
            km = k % _NP
            c = wid * (_SPW // _SB) + kd
            p0 = km * _PB
            for p_blk in range(_PB):
                scatter_desc(slot, p_blk, p0, c).start()
            if issue_next:
                build_sel(k + 1, 1 - slot)
                gather_desc(1 - slot).start()

        build_sel(0, 0)
        gather_desc(0).start()
        process(0, 0, False, True)
        process(1, 1, False, True)

        def body(t, carry):
            process(2 * t, 0, True, True)
            process(2 * t + 1, 1, True, True)
            return carry

        lax.fori_loop(1, _NCHUNK // 2 - 1, body, 0)
        process(_NCHUNK - 2, 0, True, True)
        process(_NCHUNK - 1, 1, True, False)
        for slot in range(2):
            for p_blk in range(_PB):
                scatter_desc(slot, p_blk, 0, 0).wait()

    return emb


def kernel(weight, x):
    out5 = _emb_call()(weight, x.reshape(_S * _P).astype(jnp.int32))
    return out5.transpose(2, 4, 0, 1, 3).reshape(_S, _P, _D)


# speedup vs baseline: 1.0289x; 1.0078x over previous
import functools

import jax
import jax.numpy as jnp
from jax import lax
from jax.experimental import pallas as pl
from jax.experimental.pallas import tpu as pltpu
from jax.experimental.pallas import tpu_sc as plsc

_info = plsc.get_sparse_core_info()
_NC, _NS = _info.num_cores, _info.num_subcores
_NW = _NC * _NS

_S = 16384
_P = 50
_D = 64
_SPW = _S // _NW
_SB = 128
_PB = 2
_CR = _PB * _SB
_NP = _P // _PB
_NCHUNK = (_SPW // _SB) * _NP


def _emb_call():
    mesh = plsc.VectorSubcoreMesh(core_axis_name="c", subcore_axis_name="s")

    @functools.partial(
        pl.kernel,
        mesh=mesh,
        out_type=jax.ShapeDtypeStruct((_P, 8, _S // 128, 8, 128), jnp.float32),
        scratch_types=[
            pltpu.VMEM((_SPW * _P,), jnp.int32),
            [pltpu.VMEM((_CR,), jnp.int32)] * 2,
            [pltpu.VMEM((_CR, _D), jnp.float32)] * 2,
            [pltpu.VMEM((_PB, 8, 1, 8, _SB + 1), jnp.float32)] * 2,
            [pltpu.SemaphoreType.DMA] * 2,
            [pltpu.SemaphoreType.DMA] * 2,
        ],
        compiler_params=pltpu.CompilerParams(
            use_tc_tiling_on_sc=False, needs_layout_passes=False
        ),
    )
    def emb(w_hbm, idx_hbm, out_hbm, idxall, sels, rows, tbufs, gsems, ssems):
        wid = lax.axis_index("s") * _NC + lax.axis_index("c")
        base_b = wid * (_SPW * _P)
        iota16 = lax.iota(jnp.int32, 16)
        iota50 = iota16 * _P

        pltpu.sync_copy(idx_hbm.at[pl.ds(base_b, _SPW * _P)], idxall)

        def gather_desc(slot):
            return pltpu.make_async_copy(
                w_hbm.at[sels[slot]], rows[slot], gsems[slot]
            )

        def scatter_desc(slot, p_blk, p0, c):
            return pltpu.make_async_copy(
                tbufs[slot].at[pl.ds(p_blk, 1), :, :, :, pl.ds(0, _SB)],
                out_hbm.at[pl.ds(p0 + p_blk, 1), :, pl.ds(c, 1), :, :],
                ssems[slot],
            )

        def build_sel(k, slot):
            kd = k // _NP
            km = k % _NP
            base = kd * (_SB * _P) + km * _PB
            for p_blk in range(_PB):
                for q in range(8):
                    ids = iota50 + (base + p_blk + (16 * q) * _P)
                    v = plsc.load_gather(idxall, [ids])
                    sels[slot][pl.ds(p_blk * _SB + 16 * q, 16)] = v

        rvecs = [iota16 // 8 + (16 * bi) // 8 for bi in range(4)]
        uvec = iota16 % 8
        zvec = jnp.zeros((16,), jnp.int32)
        pvecs = [jnp.full((16,), p_blk, jnp.int32) for p_blk in range(_PB)]

        def transpose_chunk(slot):
            @plsc.parallel_loop(0, _SB, unroll=8)
            def body(l):
                lvec = jnp.broadcast_to(l, (16,)).astype(jnp.int32)
                for p_blk in range(_PB):
                    r = p_blk * _SB + l
                    for bi in range(4):
                        v = rows[slot][r, pl.ds(16 * bi, 16)]
                        plsc.store_scatter(
                            tbufs[slot],
                            [pvecs[p_blk], rvecs[bi], zvec, uvec, lvec],
                            v,
                        )

        def process(k, slot, wait_prev_scatter, issue_next):
            gather_desc(slot).wait()
            if wait_prev_scatter:
                for p_blk in range(_PB):
                    scatter_desc(slot, p_blk, 0, 0).wait()
            transpose_chunk(slot)
            kd = k // _NP
            km = k % _NP
            c = wid * (_SPW // _SB) + kd
            p0 = km * _PB
            for p_blk in range(_PB):
                scatter_desc(slot, p_blk, p0, c).start()
            if issue_next:
                build_sel(k + 1, 1 - slot)
                gather_desc(1 - slot).start()

        build_sel(0, 0)
        gather_desc(0).start()
        process(0, 0, False, True)
        process(1, 1, False, True)

        def body(t, carry):
            process(2 * t, 0, True, True)
            process(2 * t + 1, 1, True, True)
            return carry

        lax.fori_loop(1, _NCHUNK // 2 - 1, body, 0)
        process(_NCHUNK - 2, 0, True, True)
        process(_NCHUNK - 1, 1, True, False)
        for slot in range(2):
            for p_blk in range(_PB):
                scatter_desc(slot, p_blk, 0, 0).wait()

    return emb


def kernel(weight, x):
    out5 = _emb_call()(weight, x.reshape(_S * _P).astype(jnp.int32))
    return out5.transpose(2, 4, 0, 1, 3).reshape(_S, _P, _D)


# speedup vs baseline: 1.1843x; 1.1511x over previous
import functools

import jax
import jax.numpy as jnp
from jax import lax
from jax.experimental import pallas as pl
from jax.experimental.pallas import tpu as pltpu
from jax.experimental.pallas import tpu_sc as plsc

_info = plsc.get_sparse_core_info()
_NC, _NS = _info.num_cores, _info.num_subcores
_NW = _NC * _NS

_S = 16384
_P = 50
_D = 64
_SPW = _S // _NW
_SB = 128
_PB = 2
_CR = _PB * _SB
_NP = _P // _PB
_NCHUNK = (_SPW // _SB) * _NP


def _emb_call():
    mesh = plsc.VectorSubcoreMesh(core_axis_name="c", subcore_axis_name="s")

    @functools.partial(
        pl.kernel,
        mesh=mesh,
        out_type=jax.ShapeDtypeStruct((_P, 8, _S // 128, 8, 128), jnp.float32),
        scratch_types=[
            pltpu.VMEM((_SPW * _P,), jnp.int32),
            [pltpu.VMEM((_CR,), jnp.int32)] * 2,
            [pltpu.VMEM((_CR, _D), jnp.float32)] * 2,
            [pltpu.VMEM((_PB, 8, 1, 8, _SB + 1), jnp.float32)] * 2,
            [pltpu.SemaphoreType.DMA] * 2,
            [pltpu.SemaphoreType.DMA] * 2,
        ],
        compiler_params=pltpu.CompilerParams(
            use_tc_tiling_on_sc=False, needs_layout_passes=False
        ),
    )
    def emb(w_hbm, idx_hbm, out_hbm, idxall, sels, rows, tbufs, gsems, ssems):
        wid = lax.axis_index("s") * _NC + lax.axis_index("c")
        base_b = wid * (_SPW * _P)
        iota16 = lax.iota(jnp.int32, 16)
        iota50 = iota16 * _P

        pltpu.sync_copy(idx_hbm.at[pl.ds(base_b, _SPW * _P)], idxall)

        def gather_desc(slot):
            return pltpu.make_async_copy(
                w_hbm.at[sels[slot]], rows[slot], gsems[slot]
            )

        def scatter_desc(slot, p_blk, p0, c):
            return pltpu.make_async_copy(
                tbufs[slot].at[pl.ds(p_blk, 1), :, :, :, pl.ds(0, _SB)],
                out_hbm.at[pl.ds(p0 + p_blk, 1), :, pl.ds(c, 1), :, :],
                ssems[slot],
            )

        def build_sel(k, slot):
            kd = k // _NP
            km = k % _NP
            base = kd * (_SB * _P) + km * _PB
            for p_blk in range(_PB):
                for q in range(8):
                    ids = iota50 + (base + p_blk + (16 * q) * _P)
                    v = plsc.load_gather(idxall, [ids])
                    sels[slot][pl.ds(p_blk * _SB + 16 * q, 16)] = v

        rvecs = [iota16 // 8 + (16 * bi) // 8 for bi in range(4)]
        uvec = iota16 % 8
        zvec = jnp.zeros((16,), jnp.int32)
        pvecs = [jnp.full((16,), p_blk, jnp.int32) for p_blk in range(_PB)]

        def transpose_chunk(slot):
            @plsc.parallel_loop(0, _SB, unroll=8)
            def body(l):
                lvec = jnp.broadcast_to(l, (16,)).astype(jnp.int32)
                for p_blk in range(_PB):
                    r = p_blk * _SB + l
                    for bi in range(4):
                        v = rows[slot][r, pl.ds(16 * bi, 16)]
                        plsc.store_scatter(
                            tbufs[slot],
                            [pvecs[p_blk], rvecs[bi], zvec, uvec, lvec],
                            v,
                        )

        def process(k, slot, wait_prev_scatter, issue_next):
            gather_desc(slot).wait()
            if issue_next:
                build_sel(k + 1, 1 - slot)
                gather_desc(1 - slot).start()
            if wait_prev_scatter:
                for p_blk in range(_PB):
                    scatter_desc(slot, p_blk, 0, 0).wait()
            transpose_chunk(slot)
            kd = k // _NP
            km = k % _NP
            c = wid * (_SPW // _SB) + kd
            p0 = km * _PB
            for p_blk in range(_PB):
                scatter_desc(slot, p_blk, p0, c).start()

        build_sel(0, 0)
        gather_desc(0).start()
        process(0, 0, False, True)
        process(1, 1, False, True)

        def body(t, carry):
            process(2 * t, 0, True, True)
            process(2 * t + 1, 1, True, True)
            return carry

        lax.fori_loop(1, _NCHUNK // 2 - 1, body, 0)
        process(_NCHUNK - 2, 0, True, True)
        process(_NCHUNK - 1, 1, True, False)
        for slot in range(2):
            for p_blk in range(_PB):
                scatter_desc(slot, p_blk, 0, 0).wait()

    return emb


def kernel(weight, x):
    out5 = _emb_call()(weight, x.reshape(_S * _P).astype(jnp.int32))
    return out5.transpose(2, 4, 0, 1, 3).reshape(_S, _P, _D)
